# baseline (speedup 1.0000x reference)
import functools

import jax
import jax.numpy as jnp
from jax import lax
from jax.experimental import pallas as pl
from jax.experimental.pallas import tpu as pltpu
from jax.experimental.pallas import tpu_sc as plsc

NUM_EMBED = 100000
EMBED_DIM = 768
OUT_DIM = 256
HALF = OUT_DIM // 2
BATCH = 4096
SEQ = 200


_MM_BLOCK = 2000


def _pack_bf16_pair(lo, hi):
    lo_bits = lax.bitcast_convert_type(
        lo.astype(jnp.bfloat16).astype(jnp.float32), jnp.int32)
    hi_bits = lax.bitcast_convert_type(
        hi.astype(jnp.bfloat16).astype(jnp.float32), jnp.int32)
    return lax.shift_right_logical(lo_bits, 16) | (
        hi_bits & jnp.int32(-65536))


def _mm_body(e_ref, wlo_ref, whi_ref, b_ref, o_ref):
    e = e_ref[...].astype(jnp.bfloat16)
    lo = (jnp.dot(e, wlo_ref[...].astype(jnp.bfloat16),
                  preferred_element_type=jnp.float32) + b_ref[0:1, :HALF])
    hi = (jnp.dot(e, whi_ref[...].astype(jnp.bfloat16),
                  preferred_element_type=jnp.float32) + b_ref[0:1, HALF:])
    o_ref[...] = _pack_bf16_pair(lo, hi)


def _project_table(emb_table, fc_wt_lo, fc_wt_hi, fc_b2d):
    return pl.pallas_call(
        _mm_body,
        grid=(NUM_EMBED // _MM_BLOCK,),
        in_specs=[
            pl.BlockSpec((_MM_BLOCK, EMBED_DIM), lambda i: (i, 0)),
            pl.BlockSpec((EMBED_DIM, HALF), lambda i: (0, 0)),
            pl.BlockSpec((EMBED_DIM, HALF), lambda i: (0, 0)),
            pl.BlockSpec((1, OUT_DIM), lambda i: (0, 0)),
        ],
        out_specs=pl.BlockSpec((_MM_BLOCK, HALF), lambda i: (i, 0)),
        out_shape=jax.ShapeDtypeStruct((NUM_EMBED, HALF), jnp.int32),
    )(emb_table, fc_wt_lo, fc_wt_hi, fc_b2d)


_NW = 32
_ROWS_PER_W = BATCH // _NW
_CHUNK = 100
_CHUNKS_PER_ROW = SEQ // _CHUNK
_CHUNKS_PER_W = _ROWS_PER_W * _CHUNKS_PER_ROW
_NG = HALF // 16
_INV_SEQ = 1.0 / SEQ


def _gather_mean_body(p_hbm, xr_hbm, out_hbm, idx_v, buf_a, buf_b, buf_c,
                      buf_d, out_v, sem_a, sem_b, sem_c, sem_d):
    wid = lax.axis_index("s") * 2 + lax.axis_index("c")

    pltpu.sync_copy(xr_hbm.at[pl.ds(wid * _CHUNKS_PER_W, _CHUNKS_PER_W)],
                    idx_v)

    def accumulate(buf, acc):
        def body(i, acc):
            acc = list(acc)
            for g in range(_NG):
                packed = buf[i, pl.ds(16 * g, 16)]
                acc[g] = acc[g] + lax.bitcast_convert_type(
                    packed << 16, jnp.float32)
                acc[_NG + g] = acc[_NG + g] + lax.bitcast_convert_type(
                    packed, jnp.float32)
            return tuple(acc)
        return lax.fori_loop(0, _CHUNK, body, acc)

    zeros = tuple(jnp.zeros((16,), jnp.float32) for _ in range(2 * _NG))

    def store_row(row, acc):
        for g in range(_NG):
            out_v[row, pl.ds(16 * g, 16)] = acc[g] * _INV_SEQ
        for g in range(_NG):
            out_v[row, pl.ds(HALF + 16 * g, 16)] = acc[_NG + g] * _INV_SEQ

    def start(c, buf, sem):
        pltpu.async_copy(p_hbm.at[idx_v.at[c]], buf, sem)

    def wait(c, buf, sem):
        pltpu.make_async_copy(p_hbm.at[idx_v.at[c]], buf, sem).wait()

    start(0, buf_a, sem_a)
    start(1, buf_b, sem_b)
    start(2, buf_c, sem_c)
    start(3, buf_d, sem_d)

    def quad_body(q, _):
        c0 = 4 * q

        wait(c0, buf_a, sem_a)
        acc = accumulate(buf_a, zeros)
        wait(c0 + 1, buf_b, sem_b)
        acc = accumulate(buf_b, acc)
        store_row(2 * q, acc)

        @pl.when(q < _ROWS_PER_W // 2 - 1)
        def _():
            start(c0 + 4, buf_a, sem_a)
            start(c0 + 5, buf_b, sem_b)

        wait(c0 + 2, buf_c, sem_c)
        acc = accumulate(buf_c, zeros)
        wait(c0 + 3, buf_d, sem_d)
        acc = accumulate(buf_d, acc)
        store_row(2 * q + 1, acc)

        @pl.when(q < _ROWS_PER_W // 2 - 1)
        def _():
            start(c0 + 6, buf_c, sem_c)
            start(c0 + 7, buf_d, sem_d)
        return 0

    lax.fori_loop(0, _ROWS_PER_W // 2, quad_body, 0)

    pltpu.sync_copy(out_v, out_hbm.at[pl.ds(wid * _ROWS_PER_W, _ROWS_PER_W)])


def _gather_mean(p, x_chunks):
    mesh = plsc.VectorSubcoreMesh(core_axis_name="c", subcore_axis_name="s")
    run = functools.partial(
        pl.kernel,
        mesh=mesh,
        out_type=jax.ShapeDtypeStruct((BATCH, OUT_DIM), jnp.float32),
        scratch_types=[
            pltpu.VMEM((_CHUNKS_PER_W, _CHUNK), jnp.int32),
            pltpu.VMEM((_CHUNK, HALF), jnp.int32),
            pltpu.VMEM((_CHUNK, HALF), jnp.int32),
            pltpu.VMEM((_CHUNK, HALF), jnp.int32),
            pltpu.VMEM((_CHUNK, HALF), jnp.int32),
            pltpu.VMEM((_ROWS_PER_W, OUT_DIM), jnp.float32),
            pltpu.SemaphoreType.DMA,
            pltpu.SemaphoreType.DMA,
            pltpu.SemaphoreType.DMA,
            pltpu.SemaphoreType.DMA,
        ],
    )(_gather_mean_body)
    return run(p, x_chunks)


def kernel(x, emb_table, fc_w, fc_b):
    fc_wt = fc_w.T
    p = _project_table(emb_table, fc_wt[:, :HALF], fc_wt[:, HALF:],
                       fc_b.reshape(1, OUT_DIM))
    x_chunks = x.astype(jnp.int32).reshape(BATCH * SEQ // _CHUNK, _CHUNK)
    return _gather_mean(p, x_chunks)


# speedup vs baseline: 1.0008x; 1.0008x over previous
import functools

import jax
import jax.numpy as jnp
from jax import lax
from jax.experimental import pallas as pl
from jax.experimental.pallas import tpu as pltpu
from jax.experimental.pallas import tpu_sc as plsc

NUM_EMBED = 100000
EMBED_DIM = 768
OUT_DIM = 256
HALF = OUT_DIM // 2
BATCH = 4096
SEQ = 200


_MM_BLOCK = 2000


def _pack_bf16_pair(lo, hi):
    lo_bits = lax.bitcast_convert_type(
        lo.astype(jnp.bfloat16).astype(jnp.float32), jnp.int32)
    hi_bits = lax.bitcast_convert_type(
        hi.astype(jnp.bfloat16).astype(jnp.float32), jnp.int32)
    return lax.shift_right_logical(lo_bits, 16) | (
        hi_bits & jnp.int32(-65536))


def _mm_body(e_ref, wlo_ref, whi_ref, b_ref, o_ref):
    e = e_ref[...]
    lo = (jnp.dot(e, wlo_ref[...], preferred_element_type=jnp.float32)
          + b_ref[0:1, :HALF])
    hi = (jnp.dot(e, whi_ref[...], preferred_element_type=jnp.float32)
          + b_ref[0:1, HALF:])
    o_ref[...] = _pack_bf16_pair(lo, hi)


def _project_table(emb_table, fc_wt_lo, fc_wt_hi, fc_b2d):
    return pl.pallas_call(
        _mm_body,
        grid=(NUM_EMBED // _MM_BLOCK,),
        in_specs=[
            pl.BlockSpec((_MM_BLOCK, EMBED_DIM), lambda i: (i, 0)),
            pl.BlockSpec((EMBED_DIM, HALF), lambda i: (0, 0)),
            pl.BlockSpec((EMBED_DIM, HALF), lambda i: (0, 0)),
            pl.BlockSpec((1, OUT_DIM), lambda i: (0, 0)),
        ],
        out_specs=pl.BlockSpec((_MM_BLOCK, HALF), lambda i: (i, 0)),
        out_shape=jax.ShapeDtypeStruct((NUM_EMBED, HALF), jnp.int32),
    )(emb_table, fc_wt_lo, fc_wt_hi, fc_b2d)


_NW = 32
_ROWS_PER_W = BATCH // _NW
_CHUNK = 100
_CHUNKS_PER_ROW = SEQ // _CHUNK
_CHUNKS_PER_W = _ROWS_PER_W * _CHUNKS_PER_ROW
_NG = HALF // 16
_INV_SEQ = 1.0 / SEQ


def _gather_mean_body(p_hbm, xr_hbm, out_hbm, idx_v, buf_a, buf_b, buf_c,
                      buf_d, out_v, sem_a, sem_b, sem_c, sem_d):
    wid = lax.axis_index("s") * 2 + lax.axis_index("c")

    pltpu.sync_copy(xr_hbm.at[pl.ds(wid * _CHUNKS_PER_W, _CHUNKS_PER_W)],
                    idx_v)

    def accumulate(buf, acc):
        def body(i, acc):
            acc = list(acc)
            for g in range(_NG):
                packed = buf[i, pl.ds(16 * g, 16)]
                acc[g] = acc[g] + lax.bitcast_convert_type(
                    packed << 16, jnp.float32)
                acc[_NG + g] = acc[_NG + g] + lax.bitcast_convert_type(
                    packed, jnp.float32)
            return tuple(acc)
        return lax.fori_loop(0, _CHUNK, body, acc)

    zeros = tuple(jnp.zeros((16,), jnp.float32) for _ in range(2 * _NG))

    def store_row(row, acc):
        for g in range(_NG):
            out_v[row, pl.ds(16 * g, 16)] = acc[g] * _INV_SEQ
        for g in range(_NG):
            out_v[row, pl.ds(HALF + 16 * g, 16)] = acc[_NG + g] * _INV_SEQ

    def start(c, buf, sem):
        pltpu.async_copy(p_hbm.at[idx_v.at[c]], buf, sem)

    def wait(c, buf, sem):
        pltpu.make_async_copy(p_hbm.at[idx_v.at[c]], buf, sem).wait()

    start(0, buf_a, sem_a)
    start(1, buf_b, sem_b)
    start(2, buf_c, sem_c)
    start(3, buf_d, sem_d)

    def quad_body(q, _):
        c0 = 4 * q

        wait(c0, buf_a, sem_a)
        acc = accumulate(buf_a, zeros)
        wait(c0 + 1, buf_b, sem_b)
        acc = accumulate(buf_b, acc)
        store_row(2 * q, acc)

        @pl.when(q < _ROWS_PER_W // 2 - 1)
        def _():
            start(c0 + 4, buf_a, sem_a)
            start(c0 + 5, buf_b, sem_b)

        wait(c0 + 2, buf_c, sem_c)
        acc = accumulate(buf_c, zeros)
        wait(c0 + 3, buf_d, sem_d)
        acc = accumulate(buf_d, acc)
        store_row(2 * q + 1, acc)

        @pl.when(q < _ROWS_PER_W // 2 - 1)
        def _():
            start(c0 + 6, buf_c, sem_c)
            start(c0 + 7, buf_d, sem_d)
        return 0

    lax.fori_loop(0, _ROWS_PER_W // 2, quad_body, 0)

    pltpu.sync_copy(out_v, out_hbm.at[pl.ds(wid * _ROWS_PER_W, _ROWS_PER_W)])


def _gather_mean(p, x_chunks):
    mesh = plsc.VectorSubcoreMesh(core_axis_name="c", subcore_axis_name="s")
    run = functools.partial(
        pl.kernel,
        mesh=mesh,
        out_type=jax.ShapeDtypeStruct((BATCH, OUT_DIM), jnp.float32),
        scratch_types=[
            pltpu.VMEM((_CHUNKS_PER_W, _CHUNK), jnp.int32),
            pltpu.VMEM((_CHUNK, HALF), jnp.int32),
            pltpu.VMEM((_CHUNK, HALF), jnp.int32),
            pltpu.VMEM((_CHUNK, HALF), jnp.int32),
            pltpu.VMEM((_CHUNK, HALF), jnp.int32),
            pltpu.VMEM((_ROWS_PER_W, OUT_DIM), jnp.float32),
            pltpu.SemaphoreType.DMA,
            pltpu.SemaphoreType.DMA,
            pltpu.SemaphoreType.DMA,
            pltpu.SemaphoreType.DMA,
        ],
    )(_gather_mean_body)
    return run(p, x_chunks)


def kernel(x, emb_table, fc_w, fc_b):
    fc_wt = fc_w.T
    p = _project_table(emb_table, fc_wt[:, :HALF], fc_wt[:, HALF:],
                       fc_b.reshape(1, OUT_DIM))
    x_chunks = x.astype(jnp.int32).reshape(BATCH * SEQ // _CHUNK, _CHUNK)
    return _gather_mean(p, x_chunks)
